# baseline (speedup 1.0000x reference)
import functools

import jax
import jax.numpy as jnp
from jax import lax
from jax.experimental import pallas as pl
from jax.experimental.pallas import tpu as pltpu
from jax.experimental.pallas import tpu_sc as plsc

H = 16
S = 2048
D = 128
M = 32768
K_NN = 16
G = 16
NG = M // G

QB = 256
MB = 2048
QT = 64

R = H * S
NW = 32
RPW = R // NW
SR = 8
NSTEP = RPW // SR


def _sims_body(q_ref, km_ref, o_ref, gm_ref):
    q = q_ref[0]
    nrm = jnp.sqrt(jnp.sum(q * q, axis=1, keepdims=True))
    qn = q / jnp.maximum(nrm, 1e-12)
    s = jax.lax.dot_general(
        qn, km_ref[0], (((1,), (1,)), ((), ())),
        preferred_element_type=jnp.float32)
    o_ref[0] = s
    gm_ref[0] = jnp.max(s.reshape(QB, MB // G, G), axis=2)


@jax.jit
def _sims(q3, km):
    return pl.pallas_call(
        _sims_body,
        grid=(H, M // MB, S // QB),
        in_specs=[
            pl.BlockSpec((1, QB, D), lambda h, im, iq: (h, iq, 0)),
            pl.BlockSpec((1, MB, D), lambda h, im, iq: (h, im, 0)),
        ],
        out_specs=[
            pl.BlockSpec((1, QB, MB), lambda h, im, iq: (h, iq, im)),
            pl.BlockSpec((1, QB, MB // G), lambda h, im, iq: (h, iq, im)),
        ],
        out_shape=[
            jax.ShapeDtypeStruct((H, S, M), jnp.float32),
            jax.ShapeDtypeStruct((H, S, NG), jnp.float32),
        ],
    )(q3, km)


def _groups_body(gm_ref, gid_ref, pay_ref):
    gm = gm_ref[0]
    h = pl.program_id(0)
    iq = pl.program_id(1)
    idx = jax.lax.broadcasted_iota(jnp.int32, (QT, NG), 1)
    t = jnp.full((QT, 1), jnp.inf, dtype=jnp.float32)
    gids = []
    for _ in range(K_NN):
        masked = jnp.where(gm < t, gm, -jnp.inf)
        m = jnp.max(masked, axis=1, keepdims=True)
        gk = jnp.min(jnp.where(masked == m, idx, jnp.int32(1 << 30)),
                     axis=1, keepdims=True)
        gids.append(gk)
        t = m
    gid = jnp.concatenate(gids, axis=1)
    rows = jax.lax.broadcasted_iota(jnp.int32, (QT, 1), 0)
    qr = h * S + iq * QT + rows
    gid_ref[0] = qr * NG + gid
    lane = jax.lax.broadcasted_iota(jnp.int32, (1, 1, G), 2)
    pay_ref[0] = (h * M + gid * G)[:, :, None] + lane


@jax.jit
def _groups(gmax16):
    return pl.pallas_call(
        _groups_body,
        grid=(H, S // QT),
        in_specs=[pl.BlockSpec((1, QT, NG), lambda h, iq: (h, iq, 0))],
        out_specs=[
            pl.BlockSpec((1, QT, K_NN), lambda h, iq: (h, iq, 0)),
            pl.BlockSpec((1, QT, K_NN, G), lambda h, iq: (h, iq, 0, 0)),
        ],
        out_shape=[
            jax.ShapeDtypeStruct((H, S, K_NN), jnp.int32),
            jax.ShapeDtypeStruct((H, S, K_NN, G), jnp.int32),
        ],
    )(gmax16)


@functools.partial(
    pl.kernel,
    mesh=plsc.VectorSubcoreMesh(core_axis_name="c", subcore_axis_name="s"),
    compiler_params=pltpu.CompilerParams(
        needs_layout_passes=False, use_tc_tiling_on_sc=False),
    out_type=[
        jax.ShapeDtypeStruct((R * K_NN, D), jnp.float32),
        jax.ShapeDtypeStruct((R * K_NN,), jnp.float32),
    ],
    scratch_types=[
        pltpu.VMEM((SR * K_NN,), jnp.int32),
        pltpu.VMEM((SR * K_NN, G), jnp.float32),
        pltpu.VMEM((SR * K_NN, G), jnp.int32),
        pltpu.VMEM((SR * K_NN,), jnp.int32),
        pltpu.VMEM((SR * K_NN,), jnp.float32),
        pltpu.VMEM((SR * K_NN, D), jnp.float32),
        pltpu.SemaphoreType.DMA,
    ],
)
def _sc_topk(cand16, gidsf, payf, vmflat, mvout, scout,
             idxb, cbuf, pbuf, vmidx, scb, mvbuf, sem):
    nc = 2
    wid = lax.axis_index("s") * nc + lax.axis_index("c")

    def step(i, carry):
        qr0 = wid * RPW + i * SR
        f0 = qr0 * K_NN
        pltpu.sync_copy(gidsf.at[pl.ds(f0, SR * K_NN)], idxb)
        pltpu.sync_copy(payf.at[pl.ds(f0, SR * K_NN)], pbuf)
        pltpu.async_copy(cand16.at[idxb], cbuf, sem).wait()
        for r in range(SR):
            run_v = None
            run_p = None
            for j in range(K_NN):
                v = cbuf[r * K_NN + j]
                p = pbuf[r * K_NN + j]
                vs, ps = plsc.sort_key_val(v, p)
                if run_v is None:
                    run_v, run_p = vs, ps
                else:
                    vr = lax.rev(vs, (0,))
                    pr = lax.rev(ps, (0,))
                    take = vr > run_v
                    vm_ = jnp.where(take, vr, run_v)
                    pm_ = jnp.where(take, pr, run_p)
                    run_v, run_p = plsc.sort_key_val(vm_, pm_)
            vmidx[pl.ds(r * K_NN, K_NN)] = run_p
            scb[pl.ds(r * K_NN, K_NN)] = run_v
        pltpu.async_copy(vmflat.at[vmidx], mvbuf, sem).wait()
        pltpu.sync_copy(mvbuf, mvout.at[pl.ds(f0, SR * K_NN)])
        pltpu.sync_copy(scb, scout.at[pl.ds(f0, SR * K_NN)])
        return carry

    lax.fori_loop(0, NSTEP, step, 0)


def _weighted_body(mv_ref, sc_ref, od_ref, g_ref, out_ref):
    w = jnp.sum(mv_ref[0] * sc_ref[0][..., None], axis=1)
    g = g_ref[0, 0, 0]
    out_ref[0] = g * w + (1.0 - g) * od_ref[0]


@jax.jit
def _weighted(mv4, sc3, outs3, gsig):
    return pl.pallas_call(
        _weighted_body,
        grid=(H, S // QT),
        in_specs=[
            pl.BlockSpec((1, QT, K_NN, D), lambda h, iq: (h, iq, 0, 0)),
            pl.BlockSpec((1, QT, K_NN), lambda h, iq: (h, iq, 0)),
            pl.BlockSpec((1, QT, D), lambda h, iq: (h, iq, 0)),
            pl.BlockSpec((1, 1, 1), lambda h, iq: (h, 0, 0)),
        ],
        out_specs=pl.BlockSpec((1, QT, D), lambda h, iq: (h, iq, 0)),
        out_shape=jax.ShapeDtypeStruct((H, S, D), jnp.float32),
    )(mv4, sc3, outs3, gsig)


def kernel(inputs, query, key, value, outputs, gate, key_memories, value_memories):
    b, h, s, d = query.shape
    q3 = jnp.transpose(query, (1, 0, 2, 3)).reshape(h, b * s, d)
    sims, gmax16 = _sims(q3, key_memories)
    gids, pay = _groups(gmax16)
    mvflat, scflat = _sc_topk(
        sims.reshape(R * NG, G),
        gids.reshape(R * K_NN),
        pay.reshape(R * K_NN, G),
        value_memories.reshape(H * M, D),
    )
    mv4 = mvflat.reshape(h, s, K_NN, d)
    sc3 = scflat.reshape(h, s, K_NN)
    gsig = jax.nn.sigmoid(gate).reshape(h, 1, 1)
    outs3 = outputs.reshape(h, s, d)
    res = _weighted(mv4, sc3, outs3, gsig)
    return res.reshape(b, h, s, d)


# speedup vs baseline: 1.0220x; 1.0220x over previous
import functools

import jax
import jax.numpy as jnp
from jax import lax
from jax.experimental import pallas as pl
from jax.experimental.pallas import tpu as pltpu
from jax.experimental.pallas import tpu_sc as plsc

H = 16
S = 2048
D = 128
M = 32768
K_NN = 16
G = 16
NG = M // G
NC128 = M // 128

QB = 256
MB = 2048
QT = 64

R = H * S
NW = 32
RPW = R // NW
SR = 8
NSTEP = RPW // SR


def _sims_body(q_ref, km_ref, o_ref, gm_ref):
    q = q_ref[0]
    nrm = jnp.sqrt(jnp.sum(q * q, axis=1, keepdims=True))
    qn = q / jnp.maximum(nrm, 1e-12)
    s = jax.lax.dot_general(
        qn, km_ref[0], (((1,), (1,)), ((), ())),
        preferred_element_type=jnp.float32)
    o_ref[...] = s.reshape(QB, MB // 128, 128)
    gm_ref[0] = jnp.max(s.reshape(QB, MB // G, G), axis=2)


@jax.jit
def _sims(q3, km):
    return pl.pallas_call(
        _sims_body,
        grid=(H, M // MB, S // QB),
        in_specs=[
            pl.BlockSpec((1, QB, D), lambda h, im, iq: (h, iq, 0)),
            pl.BlockSpec((1, MB, D), lambda h, im, iq: (h, im, 0)),
        ],
        out_specs=[
            pl.BlockSpec((QB, MB // 128, 128),
                         lambda h, im, iq: (h * (S // QB) + iq, im, 0)),
            pl.BlockSpec((1, QB, MB // G), lambda h, im, iq: (h, iq, im)),
        ],
        out_shape=[
            jax.ShapeDtypeStruct((R, NC128, 128), jnp.float32),
            jax.ShapeDtypeStruct((H, S, NG), jnp.float32),
        ],
    )(q3, km)


def _groups_body(gm_ref, gid_ref, crow_ref):
    gm = gm_ref[0]
    h = pl.program_id(0)
    iq = pl.program_id(1)
    idx = jax.lax.broadcasted_iota(jnp.int32, (QT, NG), 1)
    t = jnp.full((QT, 1), jnp.inf, dtype=jnp.float32)
    gids = []
    for _ in range(K_NN):
        masked = jnp.where(gm < t, gm, -jnp.inf)
        m = jnp.max(masked, axis=1, keepdims=True)
        gk = jnp.min(jnp.where(masked == m, idx, jnp.int32(1 << 30)),
                     axis=1, keepdims=True)
        gids.append(gk)
        t = m
    gid = jnp.concatenate(gids, axis=1)
    rows = jax.lax.broadcasted_iota(jnp.int32, (QT, 1), 0)
    qr = h * S + iq * QT + rows
    gid_ref[0] = gid
    crow_ref[0] = qr * NC128 + (gid >> 3)


@jax.jit
def _groups(gmax16):
    return pl.pallas_call(
        _groups_body,
        grid=(H, S // QT),
        in_specs=[pl.BlockSpec((1, QT, NG), lambda h, iq: (h, iq, 0))],
        out_specs=[
            pl.BlockSpec((1, QT, K_NN), lambda h, iq: (h, iq, 0)),
            pl.BlockSpec((1, QT, K_NN), lambda h, iq: (h, iq, 0)),
        ],
        out_shape=[
            jax.ShapeDtypeStruct((H, S, K_NN), jnp.int32),
            jax.ShapeDtypeStruct((H, S, K_NN), jnp.int32),
        ],
    )(gmax16)


@functools.partial(
    pl.kernel,
    mesh=plsc.VectorSubcoreMesh(core_axis_name="c", subcore_axis_name="s"),
    compiler_params=pltpu.CompilerParams(needs_layout_passes=False),
    out_type=[
        jax.ShapeDtypeStruct((R * K_NN, D), jnp.float32),
        jax.ShapeDtypeStruct((R * K_NN,), jnp.float32),
    ],
    scratch_types=[
        pltpu.VMEM((SR * K_NN,), jnp.int32),
        pltpu.VMEM((SR * K_NN,), jnp.int32),
        pltpu.VMEM((SR * K_NN, 128), jnp.float32),
        pltpu.VMEM((SR * K_NN,), jnp.int32),
        pltpu.VMEM((SR * K_NN,), jnp.float32),
        pltpu.VMEM((SR * K_NN, D), jnp.float32),
        pltpu.SemaphoreType.DMA,
    ],
)
def _sc_topk(simsrows, gidsf, crowf, vmflat, mvout, scout,
             crowb, gbuf, cbuf, vmidx, scb, mvbuf, sem):
    nc = 2
    wid = lax.axis_index("s") * nc + lax.axis_index("c")
    hh = wid // (NW // H)
    l16 = lax.iota(jnp.int32, 16)

    def step(i, carry):
        qr0 = wid * RPW + i * SR
        f0 = qr0 * K_NN
        pltpu.sync_copy(crowf.at[pl.ds(f0, SR * K_NN)], crowb)
        pltpu.sync_copy(gidsf.at[pl.ds(f0, SR * K_NN)], gbuf)
        pltpu.async_copy(simsrows.at[crowb], cbuf, sem).wait()
        for r in range(SR):
            gv = gbuf[pl.ds(r * K_NN, K_NN)]
            run_v = None
            run_p = None
            for j in range(K_NN):
                gs = jnp.max(jnp.where(l16 == j, gv, -1))
                off = (gs & 7) * G
                vals = plsc.load_gather(
                    cbuf,
                    [jnp.full((16,), r * K_NN + j, jnp.int32), off + l16])
                pay = hh * M + gs * G + l16
                vs, ps = plsc.sort_key_val(vals, pay)
                if run_v is None:
                    run_v, run_p = vs, ps
                else:
                    vr = lax.rev(vs, (0,))
                    pr = lax.rev(ps, (0,))
                    take = vr > run_v
                    vm_ = jnp.where(take, vr, run_v)
                    pm_ = jnp.where(take, pr, run_p)
                    run_v, run_p = plsc.sort_key_val(vm_, pm_)
            vmidx[pl.ds(r * K_NN, K_NN)] = run_p
            scb[pl.ds(r * K_NN, K_NN)] = run_v
        pltpu.async_copy(vmflat.at[vmidx], mvbuf, sem).wait()
        pltpu.sync_copy(mvbuf, mvout.at[pl.ds(f0, SR * K_NN)])
        pltpu.sync_copy(scb, scout.at[pl.ds(f0, SR * K_NN)])
        return carry

    lax.fori_loop(0, NSTEP, step, 0)


def _weighted_body(mv_ref, sc_ref, od_ref, g_ref, out_ref):
    w = jnp.sum(mv_ref[0] * sc_ref[0][..., None], axis=1)
    g = g_ref[0, 0, 0]
    out_ref[0] = g * w + (1.0 - g) * od_ref[0]


@jax.jit
def _weighted(mv4, sc3, outs3, gsig):
    return pl.pallas_call(
        _weighted_body,
        grid=(H, S // QT),
        in_specs=[
            pl.BlockSpec((1, QT, K_NN, D), lambda h, iq: (h, iq, 0, 0)),
            pl.BlockSpec((1, QT, K_NN), lambda h, iq: (h, iq, 0)),
            pl.BlockSpec((1, QT, D), lambda h, iq: (h, iq, 0)),
            pl.BlockSpec((1, 1, 1), lambda h, iq: (h, 0, 0)),
        ],
        out_specs=pl.BlockSpec((1, QT, D), lambda h, iq: (h, iq, 0)),
        out_shape=jax.ShapeDtypeStruct((H, S, D), jnp.float32),
    )(mv4, sc3, outs3, gsig)


def kernel(inputs, query, key, value, outputs, gate, key_memories, value_memories):
    b, h, s, d = query.shape
    q3 = jnp.transpose(query, (1, 0, 2, 3)).reshape(h, b * s, d)
    sims3, gmax16 = _sims(q3, key_memories)
    gids, crow = _groups(gmax16)
    mvflat, scflat = _sc_topk(
        sims3.reshape(R * NC128, 128),
        gids.reshape(R * K_NN),
        crow.reshape(R * K_NN),
        value_memories.reshape(H * M, D),
    )
    mv4 = mvflat.reshape(h, s, K_NN, d)
    sc3 = scflat.reshape(h, s, K_NN)
    gsig = jax.nn.sigmoid(gate).reshape(h, 1, 1)
    outs3 = outputs.reshape(h, s, d)
    res = _weighted(mv4, sc3, outs3, gsig)
    return res.reshape(b, h, s, d)


# speedup vs baseline: 3.8948x; 3.8110x over previous
import functools

import jax
import jax.numpy as jnp
from jax import lax
from jax.experimental import pallas as pl
from jax.experimental.pallas import tpu as pltpu
from jax.experimental.pallas import tpu_sc as plsc

H = 16
S = 2048
D = 128
M = 32768
K_NN = 16
G = 16
NG = M // G
NC128 = M // 128

QB = 256
MB = 2048
QT = 64

R = H * S
NW = 32
RPW = R // NW
SR = 8
NSTEP = RPW // SR


def _sims_body(q_ref, km_ref, o_ref, gm_ref):
    q = q_ref[0]
    nrm = jnp.sqrt(jnp.sum(q * q, axis=1, keepdims=True))
    qn = q / jnp.maximum(nrm, 1e-12)
    s = jax.lax.dot_general(
        qn, km_ref[0], (((1,), (1,)), ((), ())),
        preferred_element_type=jnp.float32)
    o_ref[...] = s.reshape(QB, MB // 128, 128)
    st = jax.lax.dot_general(
        km_ref[0], qn, (((1,), (1,)), ((), ())),
        preferred_element_type=jnp.float32)
    gm_ref[0] = jnp.max(st.reshape(MB // G, G, QB), axis=1)


@jax.jit
def _sims(q3, km):
    return pl.pallas_call(
        _sims_body,
        grid=(H, M // MB, S // QB),
        in_specs=[
            pl.BlockSpec((1, QB, D), lambda h, im, iq: (h, iq, 0)),
            pl.BlockSpec((1, MB, D), lambda h, im, iq: (h, im, 0)),
        ],
        out_specs=[
            pl.BlockSpec((QB, MB // 128, 128),
                         lambda h, im, iq: (h * (S // QB) + iq, im, 0)),
            pl.BlockSpec((1, MB // G, QB), lambda h, im, iq: (h, im, iq)),
        ],
        out_shape=[
            jax.ShapeDtypeStruct((R, NC128, 128), jnp.float32),
            jax.ShapeDtypeStruct((H, NG, S), jnp.float32),
        ],
    )(q3, km)


def _groups_body(gm_ref, gid_ref, crow_ref):
    gm = gm_ref[0]
    h = pl.program_id(0)
    iq = pl.program_id(1)
    idx = jax.lax.broadcasted_iota(jnp.int32, (QT, NG), 1)
    t = jnp.full((QT, 1), jnp.inf, dtype=jnp.float32)
    gids = []
    for _ in range(K_NN):
        masked = jnp.where(gm < t, gm, -jnp.inf)
        m = jnp.max(masked, axis=1, keepdims=True)
        gk = jnp.min(jnp.where(masked == m, idx, jnp.int32(1 << 30)),
                     axis=1, keepdims=True)
        gids.append(gk)
        t = m
    gid = jnp.concatenate(gids, axis=1)
    rows = jax.lax.broadcasted_iota(jnp.int32, (QT, 1), 0)
    qr = h * S + iq * QT + rows
    gid_ref[0] = gid
    crow_ref[0] = qr * NC128 + (gid >> 3)


@jax.jit
def _groups(gmax16):
    return pl.pallas_call(
        _groups_body,
        grid=(H, S // QT),
        in_specs=[pl.BlockSpec((1, QT, NG), lambda h, iq: (h, iq, 0))],
        out_specs=[
            pl.BlockSpec((1, QT, K_NN), lambda h, iq: (h, iq, 0)),
            pl.BlockSpec((1, QT, K_NN), lambda h, iq: (h, iq, 0)),
        ],
        out_shape=[
            jax.ShapeDtypeStruct((H, S, K_NN), jnp.int32),
            jax.ShapeDtypeStruct((H, S, K_NN), jnp.int32),
        ],
    )(gmax16)


@functools.partial(
    pl.kernel,
    mesh=plsc.VectorSubcoreMesh(core_axis_name="c", subcore_axis_name="s"),
    compiler_params=pltpu.CompilerParams(needs_layout_passes=False),
    out_type=[
        jax.ShapeDtypeStruct((R * K_NN, D), jnp.float32),
        jax.ShapeDtypeStruct((R * K_NN,), jnp.float32),
    ],
    scratch_types=[
        pltpu.VMEM((SR * K_NN,), jnp.int32),
        pltpu.VMEM((SR * K_NN,), jnp.int32),
        pltpu.VMEM((SR * K_NN, 128), jnp.float32),
        pltpu.VMEM((SR * K_NN,), jnp.int32),
        pltpu.VMEM((SR * K_NN,), jnp.float32),
        pltpu.VMEM((SR * K_NN, D), jnp.float32),
        pltpu.SemaphoreType.DMA,
    ],
)
def _sc_topk(simsrows, gidsf, crowf, vmflat, mvout, scout,
             crowb, gbuf, cbuf, vmidx, scb, mvbuf, sem):
    nc = 2
    wid = lax.axis_index("s") * nc + lax.axis_index("c")
    hh = wid // (NW // H)
    l16 = lax.iota(jnp.int32, 16)

    def step(i, carry):
        qr0 = wid * RPW + i * SR
        f0 = qr0 * K_NN
        pltpu.sync_copy(crowf.at[pl.ds(f0, SR * K_NN)], crowb)
        pltpu.sync_copy(gidsf.at[pl.ds(f0, SR * K_NN)], gbuf)
        pltpu.async_copy(simsrows.at[crowb], cbuf, sem).wait()
        for r in range(SR):
            gv = gbuf[pl.ds(r * K_NN, K_NN)]
            run_v = None
            run_p = None
            for j in range(K_NN):
                gs = jnp.max(jnp.where(l16 == j, gv, -1))
                off = (gs & 7) * G
                vals = plsc.load_gather(
                    cbuf,
                    [jnp.full((16,), r * K_NN + j, jnp.int32), off + l16])
                pay = hh * M + gs * G + l16
                vs, ps = plsc.sort_key_val(vals, pay)
                if run_v is None:
                    run_v, run_p = vs, ps
                else:
                    vr = lax.rev(vs, (0,))
                    pr = lax.rev(ps, (0,))
                    take = vr > run_v
                    vm_ = jnp.where(take, vr, run_v)
                    pm_ = jnp.where(take, pr, run_p)
                    run_v, run_p = plsc.sort_key_val(vm_, pm_)
            vmidx[pl.ds(r * K_NN, K_NN)] = run_p
            scb[pl.ds(r * K_NN, K_NN)] = run_v
        pltpu.async_copy(vmflat.at[vmidx], mvbuf, sem).wait()
        pltpu.sync_copy(mvbuf, mvout.at[pl.ds(f0, SR * K_NN)])
        pltpu.sync_copy(scb, scout.at[pl.ds(f0, SR * K_NN)])
        return carry

    lax.fori_loop(0, NSTEP, step, 0)


def _weighted_body(mv_ref, sc_ref, od_ref, g_ref, out_ref):
    w = jnp.sum(mv_ref[0] * sc_ref[0][..., None], axis=1)
    g = g_ref[0, 0, 0]
    out_ref[0] = g * w + (1.0 - g) * od_ref[0]


@jax.jit
def _weighted(mv4, sc3, outs3, gsig):
    return pl.pallas_call(
        _weighted_body,
        grid=(H, S // QT),
        in_specs=[
            pl.BlockSpec((1, QT, K_NN, D), lambda h, iq: (h, iq, 0, 0)),
            pl.BlockSpec((1, QT, K_NN), lambda h, iq: (h, iq, 0)),
            pl.BlockSpec((1, QT, D), lambda h, iq: (h, iq, 0)),
            pl.BlockSpec((1, 1, 1), lambda h, iq: (h, 0, 0)),
        ],
        out_specs=pl.BlockSpec((1, QT, D), lambda h, iq: (h, iq, 0)),
        out_shape=jax.ShapeDtypeStruct((H, S, D), jnp.float32),
    )(mv4, sc3, outs3, gsig)


def kernel(inputs, query, key, value, outputs, gate, key_memories, value_memories):
    b, h, s, d = query.shape
    q3 = jnp.transpose(query, (1, 0, 2, 3)).reshape(h, b * s, d)
    sims3, gmax16t = _sims(q3, key_memories)
    gids, crow = _groups(jnp.transpose(gmax16t, (0, 2, 1)))
    mvflat, scflat = _sc_topk(
        sims3.reshape(R * NC128, 128),
        gids.reshape(R * K_NN),
        crow.reshape(R * K_NN),
        value_memories.reshape(H * M, D),
    )
    mv4 = mvflat.reshape(h, s, K_NN, d)
    sc3 = scflat.reshape(h, s, K_NN)
    gsig = jax.nn.sigmoid(gate).reshape(h, 1, 1)
    outs3 = outputs.reshape(h, s, d)
    res = _weighted(mv4, sc3, outs3, gsig)
    return res.reshape(b, h, s, d)
